# Initial kernel scaffold; baseline (speedup 1.0000x reference)
import jax
import jax.numpy as jnp
from jax.experimental import pallas as pl


def kernel(entity, edge_attr, entity_neighbours, entity_embeddings, u, W, w, M):
    raise NotImplementedError("write your pallas kernel here")


# speedup vs baseline: 73.4982x; 73.4982x over previous
import functools

import jax
import jax.numpy as jnp
from jax import lax
from jax.experimental import pallas as pl
from jax.experimental.pallas import tpu as pltpu
from jax.experimental.pallas import tpu_sc as plsc

NC = 2
NS = 16
NW = NC * NS
L = 16


def _sc_gather_sum(u2, nbr_flat, entity, emb):
    V, D = u2.shape
    N = entity.shape[0]
    K = nbr_flat.shape[0] // N
    E = emb.shape[1]
    TPW = N // NW
    BCH = 128

    mesh = plsc.VectorSubcoreMesh(
        core_axis_name="c", subcore_axis_name="s",
        num_cores=NC, num_subcores=NS)

    @functools.partial(
        pl.kernel,
        out_type=(jax.ShapeDtypeStruct((N, D), jnp.float32),
                  jax.ShapeDtypeStruct((N, E), jnp.float32)),
        mesh=mesh,
        scratch_types=[
            pltpu.VMEM((TPW * K,), jnp.int32),
            pltpu.VMEM((TPW,), jnp.int32),
            pltpu.VMEM((2, K, D), jnp.float32),
            pltpu.VMEM((2, 2, D), jnp.float32),
            pltpu.VMEM((BCH, E), jnp.float32),
            pltpu.SemaphoreType.DMA,
            pltpu.SemaphoreType.DMA,
            pltpu.SemaphoreType.DMA,
            pltpu.SemaphoreType.DMA,
            pltpu.SemaphoreType.DMA,
        ],
    )
    def k(u_hbm, nbr_hbm, ent_hbm, emb_hbm, ur_hbm, base_hbm,
          idx_v, ent_v, rows_v, out_v, base_v, g0, g1, so0, so1, sb):
        wid = lax.axis_index("s") * NC + lax.axis_index("c")
        t0 = wid * TPW
        gsem = (g0, g1)
        osem = (so0, so1)

        pltpu.sync_copy(nbr_hbm.at[pl.ds(t0 * K, TPW * K)], idx_v)
        pltpu.sync_copy(ent_hbm.at[pl.ds(t0, TPW)], ent_v)

        def gather(lt, b):
            return pltpu.make_async_copy(
                u_hbm.at[idx_v.at[pl.ds(lt * K, K)]], rows_v.at[b], gsem[b])

        def out_dma(ob, row):
            return pltpu.make_async_copy(
                out_v.at[ob], ur_hbm.at[pl.ds(row, 2)], osem[ob])

        def reduce_rows(b, ob):
            def body(kk, acc):
                return tuple(acc[j] + rows_v[b, kk, pl.ds(j * L, L)]
                             for j in range(D // L))
            acc = tuple(rows_v[b, 0, pl.ds(j * L, L)] for j in range(D // L))
            acc = lax.fori_loop(1, K, body, acc)
            for j in range(D // L):
                out_v[ob, b, pl.ds(j * L, L)] = acc[j]

        gather(0, 0).start()

        def outer(cc, carry):
            for ob in range(2):
                @pl.when(cc > 0)
                def _wait_prev_out():
                    out_dma(ob, 0).wait()
                for b in range(2):
                    lt = cc * 4 + ob * 2 + b
                    nxt = jnp.minimum(lt + 1, TPW - 1)
                    gather(nxt, 1 - b).start()
                    gather(lt, b).wait()
                    reduce_rows(b, ob)
                out_dma(ob, t0 + cc * 4 + ob * 2).start()
            return carry

        lax.fori_loop(0, TPW // 4, outer, 0)

        gather(0, 0).wait()
        out_dma(0, 0).wait()
        out_dma(1, 0).wait()

        for c in range(TPW // BCH):
            pltpu.async_copy(
                emb_hbm.at[ent_v.at[pl.ds(c * BCH, BCH)]], base_v, sb).wait()
            pltpu.sync_copy(base_v, base_hbm.at[pl.ds(t0 + c * BCH, BCH)])

    return k(u2, nbr_flat, entity, emb)


def _tc_dense(ur_sum, ea, base, Wbig, w, M2, Sel, Sel2, K, R1, U, A):
    N, D = ur_sum.shape
    E = base.shape[1]
    BN = 256
    inv_k = 1.0 / K

    def body(ur_ref, ea_ref, base_ref, Wbig_ref, w_ref, M2_ref,
             Sel_ref, Sel2_ref, out_ref):
        ur = ur_ref[...] * inv_k
        eab = ea_ref[...]
        Wb = Wbig_ref[...]
        Selm = Sel_ref[...]
        earep = jnp.concatenate(
            [jnp.broadcast_to(eab[:, r:r + 1], (BN, A)) for r in range(R1)],
            axis=1)
        w_r = jnp.dot(eab, w_ref[...],
                      preferred_element_type=jnp.float32)
        wrrep = jnp.concatenate([w_r] * R1, axis=1)

        qs = []
        for r in range(R1):
            ur_r = ur[:, r * U:(r + 1) * U]
            Pr = jnp.dot(ur_r, Wb,
                         preferred_element_type=jnp.float32)
            Qr = jnp.dot(Pr * earep, Selm,
                         preferred_element_type=jnp.float32)
            qs.append(jnp.tanh(Qr))
        tq = jnp.concatenate(qs, axis=1)

        scores = jnp.dot(tq * wrrep, Sel2_ref[...],
                         preferred_element_type=jnp.float32)
        m = jnp.max(scores, axis=1, keepdims=True)
        ex = jnp.exp(scores - m)
        att = ex / jnp.sum(ex, axis=1, keepdims=True)

        vv = jnp.zeros((BN, U), jnp.float32)
        for r in range(R1):
            vv = vv + att[:, r:r + 1] * ur[:, r * U:(r + 1) * U]
        T = jnp.concatenate(
            [eab[:, r:r + 1] * vv for r in range(R1)], axis=1)
        agg = jnp.dot(T, M2_ref[...],
                      preferred_element_type=jnp.float32)
        out = base_ref[...] + agg
        nrm = jnp.sqrt(jnp.sum(out * out, axis=1, keepdims=True))
        out_ref[...] = out / jnp.maximum(nrm, 1e-12)

    grid = (N // BN,)
    fixed = lambda shape: pl.BlockSpec(shape, lambda i: (0, 0))
    return pl.pallas_call(
        body,
        grid=grid,
        in_specs=[
            pl.BlockSpec((BN, D), lambda i: (i, 0)),
            pl.BlockSpec((BN, R1), lambda i: (i, 0)),
            pl.BlockSpec((BN, E), lambda i: (i, 0)),
            fixed(Wbig.shape),
            fixed(w.shape),
            fixed(M2.shape),
            fixed(Sel.shape),
            fixed(Sel2.shape),
        ],
        out_specs=pl.BlockSpec((BN, E), lambda i: (i, 0)),
        out_shape=jax.ShapeDtypeStruct((N, E), jnp.float32),
    )(ur_sum, ea, base, Wbig, w, M2, Sel, Sel2)


def kernel(entity, edge_attr, entity_neighbours, entity_embeddings, u, W, w, M):
    N, K = entity_neighbours.shape
    V, R1, U = u.shape
    A = W.shape[2]
    E = entity_embeddings.shape[1]
    D = R1 * U

    u2 = u.reshape(V, D)
    nbr_flat = entity_neighbours.reshape(N * K)

    ur_sum, base = _sc_gather_sum(u2, nbr_flat, entity, entity_embeddings)

    Wbig = W.transpose(1, 0, 2).reshape(U, R1 * A)
    M2 = M.reshape(R1 * U, E)
    Sel = jnp.tile(jnp.eye(A, dtype=jnp.float32), (R1, 1))
    Sel2 = jnp.repeat(jnp.eye(R1, dtype=jnp.float32), A, axis=0)

    return _tc_dense(ur_sum, edge_attr, base, Wbig, w, M2, Sel, Sel2,
                     K, R1, U, A)
